# baseline (speedup 1.0000x reference)
import functools

import jax
import jax.numpy as jnp
from jax import lax
from jax.experimental import pallas as pl
from jax.experimental.pallas import tpu as pltpu
from jax.experimental.pallas import tpu_sc as plsc

NC, NS = 2, 16
NW = NC * NS
CHUNK = 64
LANES = 16


def _sc_gather_two(fp2, row_table, col_table):
  n = fp2.shape[0]
  d = row_table.shape[1]
  per_w = n // NW
  steps = per_w // CHUNK
  mesh = plsc.VectorSubcoreMesh(core_axis_name="c", subcore_axis_name="s")
  out_sds = jax.ShapeDtypeStruct((n, d), row_table.dtype)

  @functools.partial(
      pl.kernel,
      out_type=(out_sds, out_sds),
      mesh=mesh,
      compiler_params=pltpu.CompilerParams(
          use_tc_tiling_on_sc=False, needs_layout_passes=False),
      scratch_types=[
          pltpu.VMEM((per_w, 2), jnp.float32),
          pltpu.VMEM((per_w,), jnp.int32),
          pltpu.VMEM((per_w,), jnp.int32),
          pltpu.VMEM((CHUNK, d), row_table.dtype),
          pltpu.VMEM((CHUNK, d), row_table.dtype),
          pltpu.SemaphoreType.DMA,
          pltpu.SemaphoreType.DMA,
      ],
  )
  def k(fp_hbm, rt_hbm, ct_hbm, gr_hbm, gc_hbm,
        pairs_v, ir_v, ic_v, rb0, rb1, s0, s1):
    wid = lax.axis_index("s") * NC + lax.axis_index("c")
    base = wid * per_w
    pltpu.sync_copy(fp_hbm.at[pl.ds(base, per_w), pl.ds(0, 2)], pairs_v)
    zero16 = jnp.zeros((LANES,), jnp.int32)
    one16 = jnp.ones((LANES,), jnp.int32)
    for g in range(per_w // LANES):
      rows = lax.iota(jnp.int32, LANES) + (g * LANES)
      rvals = plsc.load_gather(pairs_v, [rows, zero16])
      cvals = plsc.load_gather(pairs_v, [rows, one16])
      ir_v[pl.ds(g * LANES, LANES)] = rvals.astype(jnp.int32)
      ic_v[pl.ds(g * LANES, LANES)] = cvals.astype(jnp.int32)
    items = []
    for c in range(steps):
      items.append((rt_hbm, ir_v, gr_hbm, c))
      items.append((ct_hbm, ic_v, gc_hbm, c))
    bufs, sems = (rb0, rb1), (s0, s1)
    handles = [None] * len(items)

    def start(i):
      tab, iv, _, c = items[i]
      handles[i] = pltpu.async_copy(
          tab.at[iv.at[pl.ds(c * CHUNK, CHUNK)]], bufs[i % 2], sems[i % 2])

    def finish(i):
      _, _, dst, c = items[i]
      handles[i].wait()
      pltpu.sync_copy(bufs[i % 2], dst.at[pl.ds(base + c * CHUNK, CHUNK)])

    start(0)
    for i in range(1, len(items)):
      start(i)
      finish(i - 1)
    finish(len(items) - 1)

  return k(fp2, row_table, col_table)


def _tc_body(fp_ref, w_ref, b_ref, gr_ref, gc_ref, out_ref):
  p = fp_ref[...].astype(jnp.bfloat16)
  w = w_ref[...].astype(jnp.bfloat16)
  acc = jnp.dot(p, w, preferred_element_type=jnp.float32)
  out_ref[...] = acc + b_ref[...] + gr_ref[...] + gc_ref[...]


def _tc_project_add(fp2, w_pad, b2, g_row, g_col, block_rows=1024):
  n, pw = fp2.shape
  h = w_pad.shape[1]
  grid = (n // block_rows,)
  return pl.pallas_call(
      _tc_body,
      grid=grid,
      in_specs=[
          pl.BlockSpec((block_rows, pw), lambda i: (i, 0)),
          pl.BlockSpec((pw, h), lambda i: (0, 0)),
          pl.BlockSpec((1, h), lambda i: (0, 0)),
          pl.BlockSpec((block_rows, h), lambda i: (i, 0)),
          pl.BlockSpec((block_rows, h), lambda i: (i, 0)),
      ],
      out_specs=pl.BlockSpec((block_rows, h), lambda i: (i, 0)),
      out_shape=jax.ShapeDtypeStruct((n, h), jnp.float32),
  )(fp2, w_pad, b2, g_row, g_col)


def kernel(flattened_patches, W, b, row_table, col_table):
  bsz, s, pw = flattened_patches.shape
  h = W.shape[1]
  n = bsz * s
  fp2 = flattened_patches.reshape(n, pw)
  g_row, g_col = _sc_gather_two(fp2, row_table, col_table)
  w_pad = jnp.concatenate([jnp.zeros((2, h), W.dtype), W], axis=0)
  out2 = _tc_project_add(fp2, w_pad, b.reshape(1, h), g_row, g_col)
  return out2.reshape(bsz, s, h)


# speedup vs baseline: 1.9477x; 1.9477x over previous
import functools

import jax
import jax.numpy as jnp
from jax import lax
from jax.experimental import pallas as pl
from jax.experimental.pallas import tpu as pltpu
from jax.experimental.pallas import tpu_sc as plsc

NC, NS = 2, 16
NW = NC * NS
CHUNK = 64


def _extract_body(fp_ref, ri_ref, ci_ref):
  blk = fp_ref[...]
  ri_ref[0, 0, :] = blk[:, 0].astype(jnp.int32)
  ci_ref[0, 0, :] = blk[:, 1].astype(jnp.int32)


def _tc_extract_indices(fp2, block_rows=1024):
  n = fp2.shape[0]
  g = n // block_rows
  idx_sds = jax.ShapeDtypeStruct((g, 1, block_rows), jnp.int32)
  ri, ci = pl.pallas_call(
      _extract_body,
      grid=(g,),
      in_specs=[pl.BlockSpec((block_rows, 128), lambda i: (i, 0))],
      out_specs=[
          pl.BlockSpec((1, 1, block_rows), lambda i: (i, 0, 0)),
          pl.BlockSpec((1, 1, block_rows), lambda i: (i, 0, 0)),
      ],
      out_shape=[idx_sds, idx_sds],
  )(fp2)
  return ri.reshape(n), ci.reshape(n)


def _sc_gather_two(row_idx, col_idx, row_table, col_table):
  n = row_idx.shape[0]
  d = row_table.shape[1]
  per_w = n // NW
  steps = per_w // CHUNK
  mesh = plsc.VectorSubcoreMesh(core_axis_name="c", subcore_axis_name="s")
  out_sds = jax.ShapeDtypeStruct((n, d), row_table.dtype)

  @functools.partial(
      pl.kernel,
      out_type=(out_sds, out_sds),
      mesh=mesh,
      scratch_types=[
          pltpu.VMEM((per_w,), jnp.int32),
          pltpu.VMEM((per_w,), jnp.int32),
          pltpu.VMEM((CHUNK, d), row_table.dtype),
          pltpu.VMEM((CHUNK, d), row_table.dtype),
          pltpu.SemaphoreType.DMA,
          pltpu.SemaphoreType.DMA,
      ],
  )
  def k(ri_hbm, ci_hbm, rt_hbm, ct_hbm, gr_hbm, gc_hbm,
        ir_v, ic_v, rb0, rb1, s0, s1):
    wid = lax.axis_index("s") * NC + lax.axis_index("c")
    base = wid * per_w
    pltpu.sync_copy(ri_hbm.at[pl.ds(base, per_w)], ir_v)
    pltpu.sync_copy(ci_hbm.at[pl.ds(base, per_w)], ic_v)
    items = []
    for c in range(steps):
      items.append((rt_hbm, ir_v, gr_hbm, c))
      items.append((ct_hbm, ic_v, gc_hbm, c))
    bufs, sems = (rb0, rb1), (s0, s1)
    handles = [None] * len(items)

    def start(i):
      tab, iv, _, c = items[i]
      handles[i] = pltpu.async_copy(
          tab.at[iv.at[pl.ds(c * CHUNK, CHUNK)]], bufs[i % 2], sems[i % 2])

    def finish(i):
      _, _, dst, c = items[i]
      handles[i].wait()
      pltpu.sync_copy(bufs[i % 2], dst.at[pl.ds(base + c * CHUNK, CHUNK)])

    start(0)
    for i in range(1, len(items)):
      start(i)
      finish(i - 1)
    finish(len(items) - 1)

  return k(row_idx, col_idx, row_table, col_table)


def _tc_body(fp_ref, w_ref, b_ref, gr_ref, gc_ref, out_ref):
  p = fp_ref[...].astype(jnp.bfloat16)
  w = w_ref[...].astype(jnp.bfloat16)
  acc = jnp.dot(p, w, preferred_element_type=jnp.float32)
  out_ref[...] = acc + b_ref[...] + gr_ref[...] + gc_ref[...]


def _tc_project_add(fp2, w_pad, b2, g_row, g_col, block_rows=1024):
  n, pw = fp2.shape
  h = w_pad.shape[1]
  grid = (n // block_rows,)
  return pl.pallas_call(
      _tc_body,
      grid=grid,
      in_specs=[
          pl.BlockSpec((block_rows, pw), lambda i: (i, 0)),
          pl.BlockSpec((pw, h), lambda i: (0, 0)),
          pl.BlockSpec((1, h), lambda i: (0, 0)),
          pl.BlockSpec((block_rows, h), lambda i: (i, 0)),
          pl.BlockSpec((block_rows, h), lambda i: (i, 0)),
      ],
      out_specs=pl.BlockSpec((block_rows, h), lambda i: (i, 0)),
      out_shape=jax.ShapeDtypeStruct((n, h), jnp.float32),
  )(fp2, w_pad, b2, g_row, g_col)


def kernel(flattened_patches, W, b, row_table, col_table):
  bsz, s, pw = flattened_patches.shape
  h = W.shape[1]
  n = bsz * s
  fp2 = flattened_patches.reshape(n, pw)
  row_idx, col_idx = _tc_extract_indices(fp2)
  g_row, g_col = _sc_gather_two(row_idx, col_idx, row_table, col_table)
  w_pad = jnp.concatenate([jnp.zeros((2, h), W.dtype), W], axis=0)
  out2 = _tc_project_add(fp2, w_pad, b.reshape(1, h), g_row, g_col)
  return out2.reshape(bsz, s, h)
